# Initial kernel scaffold; baseline (speedup 1.0000x reference)
import jax
import jax.numpy as jnp
from jax.experimental import pallas as pl


def kernel(feat, edge_index, adj_vals, W, b):
    raise NotImplementedError("write your pallas kernel here")


# speedup vs baseline: 8.4274x; 8.4274x over previous
import jax
import jax.numpy as jnp
from jax import lax
from jax.experimental import pallas as pl
from jax.experimental.pallas import tpu as pltpu
from jax.experimental.pallas import tpu_sc as plsc

N_NODES = 10000
N_EDGES = 320000
D = 128

NC = 2
NS = 16
NW = NC * NS
E_PER_TILE = N_EDGES // NW
CHUNK = 80
NCHUNK = E_PER_TILE // CHUNK
SUB = 25
NREFILL = NCHUNK // SUB
N_PAD = 10240
ROWS_PER_TILE = N_PAD // NS
ZROWS = CHUNK


def _sc_kernel(feat_hbm, src_hbm, dst_hbm, adj_hbm, zeros_hbm, out_hbm,
               src_v, dst_v, adj_v, rows0, rows1, agg_sh, sem0, sem1):
  c = lax.axis_index("c")
  s = lax.axis_index("s")
  wid = c * NS + s

  pltpu.sync_copy(zeros_hbm, agg_sh.at[pl.ds(s * ROWS_PER_TILE,
                                             ROWS_PER_TILE)])
  plsc.subcore_barrier()

  def start(jj, buf, sem):
    pltpu.async_copy(feat_hbm.at[src_v.at[jj]], buf, sem)

  def wait(buf, sem):
    pltpu.make_async_copy(feat_hbm.at[src_v.at[0]], buf, sem).wait()

  def process(jj, buf):
    def scale_body(g, cc):
      av = adj_v[jj, pl.ds(g * 16, 16)]
      for l in range(16):
        a = av[l]
        e = g * 16 + l
        for d in range(D // 16):
          sl = (e, pl.ds(d * 16, 16))
          buf[sl] = buf[sl] * a
      return cc

    lax.fori_loop(0, CHUNK // 16, scale_body, 0)
    pltpu.sync_copy(buf, agg_sh.at[dst_v.at[jj]], add=True)

  for r in range(NREFILL):
    pltpu.sync_copy(src_hbm.at[wid, r], src_v)
    pltpu.sync_copy(dst_hbm.at[wid, r], dst_v)
    pltpu.sync_copy(adj_hbm.at[wid, r], adj_v)

    start(0, rows0, sem0)

    def pair_body(t, carry):
      wait(rows0, sem0)
      start(2 * t + 1, rows1, sem1)
      process(2 * t, rows0)
      wait(rows1, sem1)
      start(2 * t + 2, rows0, sem0)
      process(2 * t + 1, rows1)
      return carry

    lax.fori_loop(0, (SUB - 1) // 2, pair_body, 0)
    wait(rows0, sem0)
    process(SUB - 1, rows0)
  plsc.subcore_barrier()

  for k in range(ROWS_PER_TILE // ZROWS):
    r0 = s * ROWS_PER_TILE + k * ZROWS
    pltpu.sync_copy(agg_sh.at[pl.ds(r0, ZROWS)], rows0)
    pltpu.sync_copy(rows0, out_hbm.at[c, pl.ds(r0, ZROWS)])


@jax.jit
def _sc_aggregate(feat, src, dst, adj, zeros_blk):
  mesh = plsc.VectorSubcoreMesh(core_axis_name="c", subcore_axis_name="s")
  return pl.kernel(
      _sc_kernel,
      out_type=jax.ShapeDtypeStruct((NC, N_PAD, D), jnp.float32),
      mesh=mesh,
      scratch_types=[
          pltpu.VMEM((SUB, CHUNK), jnp.int32),
          pltpu.VMEM((SUB, CHUNK), jnp.int32),
          pltpu.VMEM((SUB, CHUNK), jnp.float32),
          pltpu.VMEM((CHUNK, D), jnp.float32),
          pltpu.VMEM((CHUNK, D), jnp.float32),
          pltpu.VMEM_SHARED((N_PAD, D), jnp.float32),
          pltpu.SemaphoreType.DMA,
          pltpu.SemaphoreType.DMA,
      ],
  )(feat, src, dst, adj, zeros_blk)


def _tc_matmul_kernel(p_ref, wt_ref, b_ref, o_ref):
  x = p_ref[0] + p_ref[1]
  o_ref[...] = (
      jnp.dot(x, wt_ref[...], preferred_element_type=jnp.float32) + b_ref[...]
  )


@jax.jit
def _tc_project(partials, Wt, b2d):
  blk = 1000
  return pl.pallas_call(
      _tc_matmul_kernel,
      grid=(N_NODES // blk,),
      in_specs=[
          pl.BlockSpec((NC, blk, D), lambda i: (0, i, 0)),
          pl.BlockSpec((D, D), lambda i: (0, 0)),
          pl.BlockSpec((1, D), lambda i: (0, 0)),
      ],
      out_specs=pl.BlockSpec((blk, D), lambda i: (i, 0)),
      out_shape=jax.ShapeDtypeStruct((N_NODES, D), jnp.float32),
  )(partials, Wt, b2d)


def kernel(feat, edge_index, adj_vals, W, b):
  dst = edge_index[0].reshape(NW, NREFILL, SUB, CHUNK)
  src = edge_index[1].reshape(NW, NREFILL, SUB, CHUNK)
  adj = adj_vals.reshape(NW, NREFILL, SUB, CHUNK)
  zeros_blk = jnp.zeros((ROWS_PER_TILE, D), jnp.float32)
  partials = _sc_aggregate(feat, src, dst, adj, zeros_blk)
  return _tc_project(partials, W.T, b.reshape(1, D))
